# baseline (speedup 1.0000x reference)
import functools

import jax
import jax.numpy as jnp
from jax import lax
from jax.experimental import pallas as pl
from jax.experimental.pallas import tpu as pltpu
from jax.experimental.pallas import tpu_sc as plsc

B = 4
N = 65536
TB = N // B
C_BEV = 256
H = 200
W = 176
C_OUT = 128
HW = H * W
SCALE = 2.5
Y_OFF = 40.0

NC = 2
NS = 16
NW = NC * NS
PT = TB // NW
CH = 128
NCHUNK = PT // CH


HWB = 7040


CB = 64


def _proj_body(x_ref, w_ref, o_ref):
    c = pl.program_id(0)
    part = lax.dot_general(
        x_ref[0], w_ref[...], (((0,), (0,)), ((), ())),
        preferred_element_type=jnp.float32)

    @pl.when(c == 0)
    def _():
        o_ref[...] = part

    @pl.when(c != 0)
    def _():
        o_ref[...] += part


def _project_b(spatial3, w_bev_t, b):
    return pl.pallas_call(
        _proj_body,
        grid=(C_BEV // CB,),
        in_specs=[
            pl.BlockSpec((1, CB, HW), lambda c, b=b: (b, c, 0)),
            pl.BlockSpec((CB, C_OUT), lambda c: (c, 0)),
        ],
        out_specs=pl.BlockSpec((HW, C_OUT), lambda c: (0, 0)),
        out_shape=jax.ShapeDtypeStruct((HW, C_OUT), jnp.float32),
    )(spatial3, w_bev_t)


def _sc_body(xcol, ycol, table, out_hbm,
             x_v, y_v, ia_v, ib_v, ic_v, id_v,
             wa_v, wb_v, wc_v, wd_v, ra_v, rb_v, rc_v, rd_v, o_v, sem):
    wid = lax.axis_index("s") * NC + lax.axis_index("c")
    base = wid * PT

    def chunk(t, carry):
        off = pl.multiple_of(base + t * CH, CH)
        pltpu.sync_copy(xcol.at[pl.ds(off, CH)], x_v)
        pltpu.sync_copy(ycol.at[pl.ds(off, CH)], y_v)
        for g in range(CH // 16):
            s = pl.ds(g * 16, 16)
            x = x_v[s] * SCALE
            y = (y_v[s] + Y_OFF) * SCALE
            x0 = jnp.minimum(x.astype(jnp.int32), W - 1)
            x1 = jnp.minimum(x0 + 1, W - 1)
            y0 = jnp.minimum(y.astype(jnp.int32), H - 1)
            y1 = jnp.minimum(y0 + 1, H - 1)
            r0 = y0 * W
            r1 = y1 * W
            ia_v[s] = r0 + x0
            ic_v[s] = r0 + x1
            ib_v[s] = r1 + x0
            id_v[s] = r1 + x1
            xf0 = x0.astype(jnp.float32)
            xf1 = x1.astype(jnp.float32)
            yf0 = y0.astype(jnp.float32)
            yf1 = y1.astype(jnp.float32)
            wa_v[s] = (xf1 - x) * (yf1 - y)
            wb_v[s] = (xf1 - x) * (y - yf0)
            wc_v[s] = (x - xf0) * (yf1 - y)
            wd_v[s] = (x - xf0) * (y - yf0)
        ca = pltpu.async_copy(table.at[ia_v], ra_v, sem)
        cb = pltpu.async_copy(table.at[ib_v], rb_v, sem)
        cc = pltpu.async_copy(table.at[ic_v], rc_v, sem)
        cd = pltpu.async_copy(table.at[id_v], rd_v, sem)
        ca.wait()
        cb.wait()
        cc.wait()
        cd.wait()

        def point(j, carry2):
            sj = pl.ds(j, 16)
            wa = lax.broadcast_in_dim(wa_v[sj][0], (16,), ())
            wb = lax.broadcast_in_dim(wb_v[sj][0], (16,), ())
            wc = lax.broadcast_in_dim(wc_v[sj][0], (16,), ())
            wd = lax.broadcast_in_dim(wd_v[sj][0], (16,), ())
            for k in range(C_OUT // 16):
                sk = pl.ds(k * 16, 16)
                o_v[j, sk] = (wa * ra_v[j, sk] + wb * rb_v[j, sk]
                              + wc * rc_v[j, sk] + wd * rd_v[j, sk])
            return carry2

        lax.fori_loop(0, CH, point, 0)
        pltpu.sync_copy(o_v, out_hbm.at[pl.ds(off, CH)])
        return carry

    lax.fori_loop(0, NCHUNK, chunk, 0)


_sc_interp_b = functools.partial(
    pl.kernel,
    out_type=jax.ShapeDtypeStruct((TB, C_OUT), jnp.float32),
    mesh=plsc.VectorSubcoreMesh(core_axis_name="c", subcore_axis_name="s",
                                num_cores=NC, num_subcores=NS),
    scratch_types=[
        pltpu.VMEM((CH,), jnp.float32),
        pltpu.VMEM((CH,), jnp.float32),
        pltpu.VMEM((CH,), jnp.int32),
        pltpu.VMEM((CH,), jnp.int32),
        pltpu.VMEM((CH,), jnp.int32),
        pltpu.VMEM((CH,), jnp.int32),
        pltpu.VMEM((CH + 16,), jnp.float32),
        pltpu.VMEM((CH + 16,), jnp.float32),
        pltpu.VMEM((CH + 16,), jnp.float32),
        pltpu.VMEM((CH + 16,), jnp.float32),
        pltpu.VMEM((CH, C_OUT), jnp.float32),
        pltpu.VMEM((CH, C_OUT), jnp.float32),
        pltpu.VMEM((CH, C_OUT), jnp.float32),
        pltpu.VMEM((CH, C_OUT), jnp.float32),
        pltpu.VMEM((CH, C_OUT), jnp.float32),
        pltpu.SemaphoreType.DMA,
    ],
)(_sc_body)


PB = 2048
NPB = N // PB
BPB = TB // PB


def _m_body(b0_ref, b1_ref, b2_ref, b3_ref, c1_ref, c2_ref, c3_ref, c4_ref,
            w_ref, g_ref, bt_ref, out_ref, hs_ref, acc_ref):
    p = pl.program_id(0)
    i = pl.program_id(1)

    @pl.when(p == 0)
    def _():
        cc = jnp.concatenate(
            [c1_ref[...], c2_ref[...], c3_ref[...], c4_ref[...]], axis=1)
        hd = lax.dot_general(
            cc, w_ref[...], (((1,), (1,)), ((), ())),
            preferred_element_type=jnp.float32)
        bb = i // BPB
        sel = jnp.float32(0.0)
        for b, ref in enumerate((b0_ref, b1_ref, b2_ref, b3_ref)):
            sel = sel + jnp.where(bb == b, 1.0, 0.0) * ref[...]
        h = sel + hd
        hs_ref[pl.ds(i * PB, PB), :] = h

        @pl.when(i == 0)
        def _():
            acc_ref[...] = jnp.zeros_like(acc_ref)

        acc_ref[0:1, :] += jnp.sum(h, axis=0, keepdims=True)
        acc_ref[1:2, :] += jnp.sum(h * h, axis=0, keepdims=True)

    @pl.when(p == 1)
    def _():
        inv_n = 1.0 / N
        mean = acc_ref[0:1, :] * inv_n
        var = acc_ref[1:2, :] * inv_n - mean * mean
        scale = g_ref[...] * lax.rsqrt(var + 1e-5)
        shift = bt_ref[...] - mean * scale
        h = hs_ref[pl.ds(i * PB, PB), :]
        out_ref[...] = jnp.maximum(h * scale + shift, 0.0)


def _merged(bevs, c1, c2, c3, c4, w_conv, gamma, beta):
    def bev_map(b):
        def f(p, i, b=b):
            ii = jnp.where(p == 0, jnp.clip(i - b * BPB, 0, BPB - 1), BPB - 1)
            return (ii, 0)
        return f

    conv_map = lambda p, i: (jnp.where(p == 0, i, NPB - 1), 0)
    return pl.pallas_call(
        _m_body,
        grid=(2, NPB),
        in_specs=[
            pl.BlockSpec((PB, C_OUT), bev_map(0)),
            pl.BlockSpec((PB, C_OUT), bev_map(1)),
            pl.BlockSpec((PB, C_OUT), bev_map(2)),
            pl.BlockSpec((PB, C_OUT), bev_map(3)),
            pl.BlockSpec((PB, 16), conv_map),
            pl.BlockSpec((PB, 32), conv_map),
            pl.BlockSpec((PB, 64), conv_map),
            pl.BlockSpec((PB, 64), conv_map),
            pl.BlockSpec((C_OUT, 176), lambda p, i: (0, 0)),
            pl.BlockSpec((1, C_OUT), lambda p, i: (0, 0)),
            pl.BlockSpec((1, C_OUT), lambda p, i: (0, 0)),
        ],
        out_specs=pl.BlockSpec((PB, C_OUT),
                               lambda p, i: (jnp.where(p == 1, i, 0), 0)),
        out_shape=jax.ShapeDtypeStruct((N, C_OUT), jnp.float32),
        scratch_shapes=[
            pltpu.VMEM((N, C_OUT), jnp.float32),
            pltpu.VMEM((8, C_OUT), jnp.float32),
        ],
    )(bevs[0], bevs[1], bevs[2], bevs[3], c1, c2, c3, c4, w_conv, gamma, beta)


def kernel(point_coords, spatial_features, x_conv1, x_conv2, x_conv3, x_conv4,
           fusion_w, bn_gamma, bn_beta):
    w_bev_t = fusion_w[:, :C_BEV].T
    w_conv = fusion_w[:, C_BEV:]

    spatial3 = spatial_features.reshape(B, C_BEV, HW)
    xcol = point_coords[:, 1]
    ycol = point_coords[:, 2]

    bevs = []
    for b in range(B):
        table_b = _project_b(spatial3, w_bev_t, b)
        bevs.append(_sc_interp_b(xcol[b * TB:(b + 1) * TB],
                                 ycol[b * TB:(b + 1) * TB], table_b))

    return _merged(bevs, x_conv1, x_conv2, x_conv3, x_conv4, w_conv,
                   bn_gamma.reshape(1, C_OUT), bn_beta.reshape(1, C_OUT))


# speedup vs baseline: 1.0992x; 1.0992x over previous
import functools

import jax
import jax.numpy as jnp
from jax import lax
from jax.experimental import pallas as pl
from jax.experimental.pallas import tpu as pltpu
from jax.experimental.pallas import tpu_sc as plsc

B = 4
N = 65536
TB = N // B
C_BEV = 256
H = 200
W = 176
C_OUT = 128
HW = H * W
SCALE = 2.5
Y_OFF = 40.0

NC = 2
NS = 16
NW = NC * NS
PT = TB // NW
CH = 128
NCHUNK = PT // CH


HWB = 7040


def _proj_body(x_ref, w_ref, o_ref):
    x = x_ref[0]
    o_ref[...] = lax.dot_general(
        x, w_ref[...], (((0,), (0,)), ((), ())),
        preferred_element_type=jnp.float32)


def _project_b(spatial3, w_bev_t, b):
    return pl.pallas_call(
        _proj_body,
        grid=(HW // HWB,),
        in_specs=[
            pl.BlockSpec((1, C_BEV, HWB), lambda j, b=b: (b, 0, j)),
            pl.BlockSpec((C_BEV, C_OUT), lambda j: (0, 0)),
        ],
        out_specs=pl.BlockSpec((HWB, C_OUT), lambda j: (j, 0)),
        out_shape=jax.ShapeDtypeStruct((HW, C_OUT), jnp.float32),
    )(spatial3, w_bev_t)


CH2 = 64
NCHUNK2 = PT // CH2


def _sc_body(xcol, ycol, table, out_hbm,
             x0_v, y0_v, ia0_v, ib0_v, ic0_v, id0_v,
             wa0_v, wb0_v, wc0_v, wd0_v, ra0_v, rb0_v, rc0_v, rd0_v,
             x1_v, y1_v, ia1_v, ib1_v, ic1_v, id1_v,
             wa1_v, wb1_v, wc1_v, wd1_v, ra1_v, rb1_v, rc1_v, rd1_v,
             o_v, sem0, sem1):
    wid = lax.axis_index("s") * NC + lax.axis_index("c")
    base = wid * PT
    sets = (
        (x0_v, y0_v, ia0_v, ib0_v, ic0_v, id0_v,
         wa0_v, wb0_v, wc0_v, wd0_v, ra0_v, rb0_v, rc0_v, rd0_v, sem0),
        (x1_v, y1_v, ia1_v, ib1_v, ic1_v, id1_v,
         wa1_v, wb1_v, wc1_v, wd1_v, ra1_v, rb1_v, rc1_v, rd1_v, sem1),
    )

    def stage(t, p):
        (x_v, y_v, ia_v, ib_v, ic_v, id_v,
         wa_v, wb_v, wc_v, wd_v, ra_v, rb_v, rc_v, rd_v, sem) = sets[p]
        off = pl.multiple_of(base + t * CH2, CH2)
        pltpu.sync_copy(xcol.at[pl.ds(off, CH2)], x_v)
        pltpu.sync_copy(ycol.at[pl.ds(off, CH2)], y_v)
        for g in range(CH2 // 16):
            s = pl.ds(g * 16, 16)
            x = x_v[s] * SCALE
            y = (y_v[s] + Y_OFF) * SCALE
            x0 = jnp.minimum(x.astype(jnp.int32), W - 1)
            x1 = jnp.minimum(x0 + 1, W - 1)
            y0 = jnp.minimum(y.astype(jnp.int32), H - 1)
            y1 = jnp.minimum(y0 + 1, H - 1)
            r0 = y0 * W
            r1 = y1 * W
            ia_v[s] = r0 + x0
            ic_v[s] = r0 + x1
            ib_v[s] = r1 + x0
            id_v[s] = r1 + x1
            xf0 = x0.astype(jnp.float32)
            xf1 = x1.astype(jnp.float32)
            yf0 = y0.astype(jnp.float32)
            yf1 = y1.astype(jnp.float32)
            wa_v[s] = (xf1 - x) * (yf1 - y)
            wb_v[s] = (xf1 - x) * (y - yf0)
            wc_v[s] = (x - xf0) * (yf1 - y)
            wd_v[s] = (x - xf0) * (y - yf0)
        pltpu.async_copy(table.at[ia_v], ra_v, sem)
        pltpu.async_copy(table.at[ib_v], rb_v, sem)
        pltpu.async_copy(table.at[ic_v], rc_v, sem)
        pltpu.async_copy(table.at[id_v], rd_v, sem)

    def consume(t, p):
        (x_v, y_v, ia_v, ib_v, ic_v, id_v,
         wa_v, wb_v, wc_v, wd_v, ra_v, rb_v, rc_v, rd_v, sem) = sets[p]
        off = pl.multiple_of(base + t * CH2, CH2)
        pltpu.make_async_copy(table.at[ia_v], ra_v, sem).wait()
        pltpu.make_async_copy(table.at[ib_v], rb_v, sem).wait()
        pltpu.make_async_copy(table.at[ic_v], rc_v, sem).wait()
        pltpu.make_async_copy(table.at[id_v], rd_v, sem).wait()

        def point(j, carry2):
            sj = pl.ds(j, 16)
            wa = lax.broadcast_in_dim(wa_v[sj][0], (16,), ())
            wb = lax.broadcast_in_dim(wb_v[sj][0], (16,), ())
            wc = lax.broadcast_in_dim(wc_v[sj][0], (16,), ())
            wd = lax.broadcast_in_dim(wd_v[sj][0], (16,), ())
            for k in range(C_OUT // 16):
                sk = pl.ds(k * 16, 16)
                o_v[j, sk] = (wa * ra_v[j, sk] + wb * rb_v[j, sk]
                              + wc * rc_v[j, sk] + wd * rd_v[j, sk])
            return carry2

        lax.fori_loop(0, CH2, point, 0)
        pltpu.sync_copy(o_v, out_hbm.at[pl.ds(off, CH2)])

    stage(0, 0)

    def pair(q, carry):
        t0 = q * 2
        stage(t0 + 1, 1)
        consume(t0, 0)

        @pl.when(t0 + 2 < NCHUNK2)
        def _():
            stage(t0 + 2, 0)

        consume(t0 + 1, 1)
        return carry

    lax.fori_loop(0, NCHUNK2 // 2, pair, 0)


def _sc_scratch_set():
    return [
        pltpu.VMEM((CH2,), jnp.float32),
        pltpu.VMEM((CH2,), jnp.float32),
        pltpu.VMEM((CH2,), jnp.int32),
        pltpu.VMEM((CH2,), jnp.int32),
        pltpu.VMEM((CH2,), jnp.int32),
        pltpu.VMEM((CH2,), jnp.int32),
        pltpu.VMEM((CH2 + 16,), jnp.float32),
        pltpu.VMEM((CH2 + 16,), jnp.float32),
        pltpu.VMEM((CH2 + 16,), jnp.float32),
        pltpu.VMEM((CH2 + 16,), jnp.float32),
        pltpu.VMEM((CH2, C_OUT), jnp.float32),
        pltpu.VMEM((CH2, C_OUT), jnp.float32),
        pltpu.VMEM((CH2, C_OUT), jnp.float32),
        pltpu.VMEM((CH2, C_OUT), jnp.float32),
    ]


_sc_interp_b = functools.partial(
    pl.kernel,
    out_type=jax.ShapeDtypeStruct((TB, C_OUT), jnp.float32),
    mesh=plsc.VectorSubcoreMesh(core_axis_name="c", subcore_axis_name="s",
                                num_cores=NC, num_subcores=NS),
    scratch_types=_sc_scratch_set() + _sc_scratch_set() + [
        pltpu.VMEM((CH2, C_OUT), jnp.float32),
        pltpu.SemaphoreType.DMA,
        pltpu.SemaphoreType.DMA,
    ],
)(_sc_body)


PB = 2048
NPB = N // PB
BPB = TB // PB


def _m_body(b0_ref, b1_ref, b2_ref, b3_ref, c1_ref, c2_ref, c3_ref, c4_ref,
            w_ref, g_ref, bt_ref, out_ref, hs_ref, acc_ref):
    p = pl.program_id(0)
    i = pl.program_id(1)

    @pl.when(p == 0)
    def _():
        cc = jnp.concatenate(
            [c1_ref[...], c2_ref[...], c3_ref[...], c4_ref[...]], axis=1)
        hd = lax.dot_general(
            cc, w_ref[...], (((1,), (1,)), ((), ())),
            preferred_element_type=jnp.float32)
        bb = i // BPB
        sel = jnp.float32(0.0)
        for b, ref in enumerate((b0_ref, b1_ref, b2_ref, b3_ref)):
            sel = sel + jnp.where(bb == b, 1.0, 0.0) * ref[...]
        h = sel + hd
        hs_ref[pl.ds(i * PB, PB), :] = h

        @pl.when(i == 0)
        def _():
            acc_ref[...] = jnp.zeros_like(acc_ref)

        acc_ref[0:1, :] += jnp.sum(h, axis=0, keepdims=True)
        acc_ref[1:2, :] += jnp.sum(h * h, axis=0, keepdims=True)

    @pl.when(p == 1)
    def _():
        inv_n = 1.0 / N
        mean = acc_ref[0:1, :] * inv_n
        var = acc_ref[1:2, :] * inv_n - mean * mean
        scale = g_ref[...] * lax.rsqrt(var + 1e-5)
        shift = bt_ref[...] - mean * scale
        h = hs_ref[pl.ds(i * PB, PB), :]
        out_ref[...] = jnp.maximum(h * scale + shift, 0.0)


def _merged(bevs, c1, c2, c3, c4, w_conv, gamma, beta):
    def bev_map(b):
        def f(p, i, b=b):
            ii = jnp.where(p == 0, jnp.clip(i - b * BPB, 0, BPB - 1), BPB - 1)
            return (ii, 0)
        return f

    conv_map = lambda p, i: (jnp.where(p == 0, i, NPB - 1), 0)
    return pl.pallas_call(
        _m_body,
        grid=(2, NPB),
        in_specs=[
            pl.BlockSpec((PB, C_OUT), bev_map(0)),
            pl.BlockSpec((PB, C_OUT), bev_map(1)),
            pl.BlockSpec((PB, C_OUT), bev_map(2)),
            pl.BlockSpec((PB, C_OUT), bev_map(3)),
            pl.BlockSpec((PB, 16), conv_map),
            pl.BlockSpec((PB, 32), conv_map),
            pl.BlockSpec((PB, 64), conv_map),
            pl.BlockSpec((PB, 64), conv_map),
            pl.BlockSpec((C_OUT, 176), lambda p, i: (0, 0)),
            pl.BlockSpec((1, C_OUT), lambda p, i: (0, 0)),
            pl.BlockSpec((1, C_OUT), lambda p, i: (0, 0)),
        ],
        out_specs=pl.BlockSpec((PB, C_OUT),
                               lambda p, i: (jnp.where(p == 1, i, 0), 0)),
        out_shape=jax.ShapeDtypeStruct((N, C_OUT), jnp.float32),
        scratch_shapes=[
            pltpu.VMEM((N, C_OUT), jnp.float32),
            pltpu.VMEM((8, C_OUT), jnp.float32),
        ],
    )(bevs[0], bevs[1], bevs[2], bevs[3], c1, c2, c3, c4, w_conv, gamma, beta)


def kernel(point_coords, spatial_features, x_conv1, x_conv2, x_conv3, x_conv4,
           fusion_w, bn_gamma, bn_beta):
    w_bev_t = fusion_w[:, :C_BEV].T
    w_conv = fusion_w[:, C_BEV:]

    spatial3 = spatial_features.reshape(B, C_BEV, HW)
    xcol = point_coords[:, 1]
    ycol = point_coords[:, 2]

    bevs = []
    for b in range(B):
        table_b = _project_b(spatial3, w_bev_t, b)
        bevs.append(_sc_interp_b(xcol[b * TB:(b + 1) * TB],
                                 ycol[b * TB:(b + 1) * TB], table_b))

    return _merged(bevs, x_conv1, x_conv2, x_conv3, x_conv4, w_conv,
                   bn_gamma.reshape(1, C_OUT), bn_beta.reshape(1, C_OUT))
